# baseline (speedup 1.0000x reference)
import functools
import jax
import jax.numpy as jnp
from jax import lax
from jax.experimental import pallas as pl
from jax.experimental.pallas import tpu as pltpu

N_GRID = 648
N_MESH = 5882
E_PROC = N_MESH * 6
FEAT = 78
ND = 128
ED = 128
HDD = 64

NM_PAD = 5888
EP_PAD = 36864
NG_PAD = 768

_MT = 736
_ET = 2304


def _ln(h, g, be):
    mu = jnp.mean(h, axis=-1, keepdims=True)
    v = jnp.mean((h - mu) * (h - mu), axis=-1, keepdims=True)
    return (h - mu) * lax.rsqrt(v + 1e-5) * g + be


def _ea_body(x_ref, a0_ref, a1_ref, wna_ref, wnb_ref, bn1_ref, wn2_ref,
             bn2_ref, g_ref, be_ref, wa_ref, wb_ref,
             xn_ref, xa_ref, xb_ref):
    agg = a0_ref[...] + a1_ref[...]
    x = x_ref[...]
    h = jnp.maximum(
        jnp.dot(x, wna_ref[...], preferred_element_type=jnp.float32)
        + jnp.dot(agg, wnb_ref[...], preferred_element_type=jnp.float32)
        + bn1_ref[...], 0.0)
    h2 = jnp.dot(h, wn2_ref[...], preferred_element_type=jnp.float32) + bn2_ref[...]
    xn = x + _ln(h2, g_ref[...], be_ref[...])
    xn_ref[...] = xn
    xa_ref[...] = jnp.dot(xn, wa_ref[...], preferred_element_type=jnp.float32)
    xb_ref[...] = jnp.dot(xn, wb_ref[...], preferred_element_type=jnp.float32)


def _ea_call(x, a0, a1, pn, wa_next, wb_next):
    wn1 = pn['l1']['w']
    row = lambda i, j: pl.BlockSpec((_MT, 128), lambda k: (k, 0))
    full = pl.BlockSpec((128, 128), lambda k: (0, 0))
    vec = pl.BlockSpec((1, 128), lambda k: (0, 0))
    out_sh = jax.ShapeDtypeStruct((NM_PAD, 128), jnp.float32)
    return pl.pallas_call(
        _ea_body,
        grid=(NM_PAD // _MT,),
        in_specs=[row(0, 0), row(0, 0), row(0, 0), full, full, vec, full,
                  vec, vec, vec, full, full],
        out_specs=[row(0, 0), row(0, 0), row(0, 0)],
        out_shape=[out_sh, out_sh, out_sh],
    )(x, a0, a1, wn1[:ND], wn1[ND:], pn['l1']['b'][None], pn['l2']['w'],
      pn['l2']['b'][None], pn['g'][None], pn['be'][None], wa_next, wb_next)


def _c_body(s_ref, e_ref, wc_ref, b1_ref, w2_ref, b2_ref, g_ref, be_ref,
            out_ref):
    e = e_ref[...]
    ec = jnp.dot(e, wc_ref[...], preferred_element_type=jnp.float32) + b1_ref[...]
    h1 = jnp.maximum(s_ref[...] + ec, 0.0)
    h2 = jnp.dot(h1, w2_ref[...], preferred_element_type=jnp.float32) + b2_ref[...]
    out_ref[...] = e + _ln(h2, g_ref[...], be_ref[...])


def _c_call(s, e, pe):
    w1 = pe['l1']['w']
    row = pl.BlockSpec((_ET, 128), lambda k: (k, 0))
    full = pl.BlockSpec((128, 128), lambda k: (0, 0))
    vec = pl.BlockSpec((1, 128), lambda k: (0, 0))
    return pl.pallas_call(
        _c_body,
        grid=(EP_PAD // _ET,),
        in_specs=[row, row, full, vec, full, vec, vec, vec],
        out_specs=row,
        out_shape=jax.ShapeDtypeStruct((EP_PAD, 128), jnp.float32),
    )(s, e, w1[2 * ND:], pe['l1']['b'][None], pe['l2']['w'],
      pe['l2']['b'][None], pe['g'][None], pe['be'][None])


def _ef_body(f_ref, w1_ref, b1_ref, w2_ref, b2_ref, g_ref, be_ref, out_ref):
    h = jnp.maximum(
        jnp.dot(f_ref[...], w1_ref[...], preferred_element_type=jnp.float32)
        + b1_ref[...], 0.0)
    h2 = jnp.dot(h, w2_ref[...], preferred_element_type=jnp.float32) + b2_ref[...]
    out_ref[...] = _ln(h2, g_ref[...], be_ref[...])


def _ef_call(pf_pad, p):
    w1 = jnp.zeros((8, 128), jnp.float32).at[:3].set(p['l1']['w'])
    row_in = pl.BlockSpec((_ET, 8), lambda k: (k, 0))
    row_out = pl.BlockSpec((_ET, 128), lambda k: (k, 0))
    vec = pl.BlockSpec((1, 128), lambda k: (0, 0))
    return pl.pallas_call(
        _ef_body,
        grid=(EP_PAD // _ET,),
        in_specs=[row_in, pl.BlockSpec((8, 128), lambda k: (0, 0)), vec,
                  pl.BlockSpec((128, 128), lambda k: (0, 0)), vec, vec, vec],
        out_specs=row_out,
        out_shape=jax.ShapeDtypeStruct((EP_PAD, 128), jnp.float32),
    )(pf_pad, w1, p['l1']['b'][None], p['l2']['w'], p['l2']['b'][None],
      p['g'][None], p['be'][None])


def _enc1_body(f_ref, ef_ref,
               nw1_ref, nb1_ref, nw2_ref, nb2_ref, ng_ref, nbe_ref,
               ew1_ref, eb1_ref, ew2_ref, eb2_ref, eg_ref, ebe_ref,
               wa_ref, wc_ref, gb1_ref, gw2_ref, gb2_ref, gg_ref, gbe_ref,
               xg_ref, m_ref):
    h = jnp.maximum(
        jnp.dot(f_ref[...], nw1_ref[...], preferred_element_type=jnp.float32)
        + nb1_ref[...], 0.0)
    xg = _ln(jnp.dot(h, nw2_ref[...], preferred_element_type=jnp.float32)
             + nb2_ref[...], ng_ref[...], nbe_ref[...])
    xg_ref[...] = xg
    h = jnp.maximum(
        jnp.dot(ef_ref[...], ew1_ref[...], preferred_element_type=jnp.float32)
        + eb1_ref[...], 0.0)
    ee = _ln(jnp.dot(h, ew2_ref[...], preferred_element_type=jnp.float32)
             + eb2_ref[...], eg_ref[...], ebe_ref[...])
    h = jnp.maximum(
        jnp.dot(xg, wa_ref[...], preferred_element_type=jnp.float32)
        + jnp.dot(ee, wc_ref[...], preferred_element_type=jnp.float32)
        + gb1_ref[...], 0.0)
    m_ref[...] = _ln(jnp.dot(h, gw2_ref[...], preferred_element_type=jnp.float32)
                     + gb2_ref[...], gg_ref[...], gbe_ref[...])


def _enc1_call(feats_p, enc_ef_p, params):
    pn, pe, pg = params['enc_node'], params['enc_edge'], params['enc_gn_e']
    nw1 = jnp.zeros((80, 128), jnp.float32).at[:FEAT].set(pn['l1']['w'])
    ew1 = jnp.zeros((8, 128), jnp.float32).at[:3].set(pe['l1']['w'])
    gw1 = pg['l1']['w']
    nb = pl.BlockSpec(None, lambda: (0, 0))
    out_sh = jax.ShapeDtypeStruct((N_GRID, 128), jnp.float32)
    return pl.pallas_call(
        _enc1_body,
        in_specs=[nb] * 21,
        out_specs=[nb, nb],
        out_shape=[out_sh, out_sh],
    )(feats_p, enc_ef_p,
      nw1, pn['l1']['b'][None], pn['l2']['w'], pn['l2']['b'][None],
      pn['g'][None], pn['be'][None],
      ew1, pe['l1']['b'][None], pe['l2']['w'], pe['l2']['b'][None],
      pe['g'][None], pe['be'][None],
      gw1[:ND], gw1[2 * ND:], pg['l1']['b'][None], pg['l2']['w'],
      pg['l2']['b'][None], pg['g'][None], pg['be'][None])


def _dec_body(xg_ref, gd_ref, ef_ref, f_ref,
              ew1_ref, eb1_ref, ew2_ref, eb2_ref, eg_ref, ebe_ref,
              wb_ref, wc_ref, gb1_ref, gw2_ref, gb2_ref, gg_ref, gbe_ref,
              wna_ref, wnb_ref, nb1_ref, nw2_ref, nb2_ref, ng_ref, nbe_ref,
              ow1_ref, ob1_ref, ow2_ref, ob2_ref,
              out_ref):
    xg = xg_ref[...]
    h = jnp.maximum(
        jnp.dot(ef_ref[...], ew1_ref[...], preferred_element_type=jnp.float32)
        + eb1_ref[...], 0.0)
    ed = _ln(jnp.dot(h, ew2_ref[...], preferred_element_type=jnp.float32)
             + eb2_ref[...], eg_ref[...], ebe_ref[...])
    h = jnp.maximum(
        gd_ref[...]
        + jnp.dot(xg, wb_ref[...], preferred_element_type=jnp.float32)
        + jnp.dot(ed, wc_ref[...], preferred_element_type=jnp.float32)
        + gb1_ref[...], 0.0)
    m = _ln(jnp.dot(h, gw2_ref[...], preferred_element_type=jnp.float32)
            + gb2_ref[...], gg_ref[...], gbe_ref[...])
    h = jnp.maximum(
        jnp.dot(xg, wna_ref[...], preferred_element_type=jnp.float32)
        + jnp.dot(m, wnb_ref[...], preferred_element_type=jnp.float32)
        + nb1_ref[...], 0.0)
    x_out = _ln(jnp.dot(h, nw2_ref[...], preferred_element_type=jnp.float32)
                + nb2_ref[...], ng_ref[...], nbe_ref[...])
    d1 = jnp.maximum(
        jnp.dot(x_out, ow1_ref[...], preferred_element_type=jnp.float32)
        + ob1_ref[...], 0.0)
    delta = jnp.dot(d1, ow2_ref[...], preferred_element_type=jnp.float32) + ob2_ref[...]
    out_ref[...] = f_ref[...] + delta


def _dec_call(x_grid, gdec, dec_ef_p, feats_p, params):
    pe, pg, pn, po = (params['dec_edge'], params['dec_gn_e'],
                      params['dec_gn_n'], params['dec_out'])
    ew1 = jnp.zeros((8, 128), jnp.float32).at[:3].set(pe['l1']['w'])
    gw1 = pg['l1']['w']
    nw1 = pn['l1']['w']
    ow2 = jnp.zeros((HDD, 80), jnp.float32).at[:, :FEAT].set(po['l2']['w'])
    ob2 = jnp.zeros((1, 80), jnp.float32).at[0, :FEAT].set(po['l2']['b'])
    nb = pl.BlockSpec(None, lambda: (0, 0))
    return pl.pallas_call(
        _dec_body,
        in_specs=[nb] * 28,
        out_specs=nb,
        out_shape=jax.ShapeDtypeStruct((N_GRID, 80), jnp.float32),
    )(x_grid, gdec, dec_ef_p, feats_p,
      ew1, pe['l1']['b'][None], pe['l2']['w'], pe['l2']['b'][None],
      pe['g'][None], pe['be'][None],
      gw1[ND:2 * ND], gw1[2 * ND:], pg['l1']['b'][None], pg['l2']['w'],
      pg['l2']['b'][None], pg['g'][None], pg['be'][None],
      nw1[:ND], nw1[ND:], pn['l1']['b'][None], pn['l2']['w'],
      pn['l2']['b'][None], pn['g'][None], pn['be'][None],
      po['l1']['w'], po['l1']['b'][None], ow2, ob2)


def _gather_sum(xa, xb, ps_g, pd_g):
    return xa[ps_g] + xb[pd_g]


def _scatter_add(e, idx_s, nrows):
    agg = jnp.zeros((nrows, 128), jnp.float32).at[idx_s].add(e)
    z = jnp.zeros((nrows, 128), jnp.float32)
    return agg, z


@jax.jit
def _run(features, params, enc_ef, proc_ef, dec_ef, enc_edges, proc_edges,
         dec_edges):
    feats_p = jnp.zeros((N_GRID, 80), jnp.float32).at[:, :FEAT].set(
        features.reshape(N_GRID, FEAT))
    enc_ef_p = jnp.zeros((N_GRID, 8), jnp.float32).at[:, :3].set(enc_ef)
    dec_ef_p = jnp.zeros((N_GRID, 8), jnp.float32).at[:, :3].set(dec_ef)
    pf_pad = jnp.zeros((EP_PAD, 8), jnp.float32).at[:E_PROC, :3].set(proc_ef)

    ps = proc_edges[0]
    pd = proc_edges[1]
    ps_g = jnp.pad(ps, (0, EP_PAD - E_PROC))
    pd_g = jnp.pad(pd, (0, EP_PAD - E_PROC))
    pd_s = jnp.pad(pd, (0, EP_PAD - E_PROC), constant_values=N_MESH)
    enc_d = jnp.pad(enc_edges[1], (0, NG_PAD - N_GRID), constant_values=N_MESH)
    ds = jnp.pad(dec_edges[0], (0, NG_PAD - N_GRID))

    x_grid, m = _enc1_call(feats_p, enc_ef_p, params)
    m_pad = jnp.zeros((NG_PAD, 128), jnp.float32).at[:N_GRID].set(m)
    a0, a1 = _scatter_add(m_pad, enc_d, NM_PAD)
    blk0 = params['blocks'][0]
    w1n = blk0['e']['l1']['w']
    zeros_x = jnp.zeros((NM_PAD, 128), jnp.float32)
    x, xa, xb = _ea_call(zeros_x, a0, a1, params['enc_gn_n'],
                         w1n[:ND], w1n[ND:2 * ND])

    e = _ef_call(pf_pad, params['proc_edge'])

    nb = len(params['blocks'])
    gw1 = params['dec_gn_e']['l1']['w']
    for k, blk in enumerate(params['blocks']):
        s = _gather_sum(xa, xb, ps_g, pd_g)
        e = _c_call(s, e, blk['e'])
        a0, a1 = _scatter_add(e, pd_s, NM_PAD)
        if k + 1 < nb:
            w1n = params['blocks'][k + 1]['e']['l1']['w']
            wa_next, wb_next = w1n[:ND], w1n[ND:2 * ND]
        else:
            wa_next, wb_next = gw1[:ND], gw1[ND:2 * ND]
        x, xa, xb = _ea_call(x, a0, a1, blk['n'], wa_next, wb_next)

    gdec = xa[ds][:N_GRID]
    out = _dec_call(x_grid, gdec, dec_ef_p, feats_p, params)
    return out[:, :FEAT].reshape(1, N_GRID, FEAT)


def kernel(features, params, enc_ef, proc_ef, dec_ef, enc_edges, proc_edges,
           dec_edges):
    return _run(features, params, enc_ef, proc_ef, dec_ef, enc_edges,
                proc_edges, dec_edges)


# speedup vs baseline: 1.9934x; 1.9934x over previous
import functools
import jax
import jax.numpy as jnp
from jax import lax
from jax.experimental import pallas as pl
from jax.experimental.pallas import tpu as pltpu
from jax.experimental.pallas import tpu_sc as plsc

N_GRID = 648
N_MESH = 5882
E_PROC = N_MESH * 6
FEAT = 78
ND = 128
ED = 128
HDD = 64

NM_PAD = 5888
EP_PAD = 36864
NG_PAD = 4096

_MT = 736
_ET = 2304


def _ln(h, g, be):
    mu = jnp.mean(h, axis=-1, keepdims=True)
    v = jnp.mean((h - mu) * (h - mu), axis=-1, keepdims=True)
    return (h - mu) * lax.rsqrt(v + 1e-5) * g + be


def _ea_body(x_ref, a0_ref, a1_ref, wna_ref, wnb_ref, bn1_ref, wn2_ref,
             bn2_ref, g_ref, be_ref, wa_ref, wb_ref,
             xn_ref, xa_ref, xb_ref):
    agg = a0_ref[...] + a1_ref[...]
    x = x_ref[...]
    h = jnp.maximum(
        jnp.dot(x, wna_ref[...], preferred_element_type=jnp.float32)
        + jnp.dot(agg, wnb_ref[...], preferred_element_type=jnp.float32)
        + bn1_ref[...], 0.0)
    h2 = jnp.dot(h, wn2_ref[...], preferred_element_type=jnp.float32) + bn2_ref[...]
    xn = x + _ln(h2, g_ref[...], be_ref[...])
    xn_ref[...] = xn
    xa_ref[...] = jnp.dot(xn, wa_ref[...], preferred_element_type=jnp.float32)
    xb_ref[...] = jnp.dot(xn, wb_ref[...], preferred_element_type=jnp.float32)


def _ea_call(x, a0, a1, pn, wa_next, wb_next):
    wn1 = pn['l1']['w']
    row = lambda i, j: pl.BlockSpec((_MT, 128), lambda k: (k, 0))
    full = pl.BlockSpec((128, 128), lambda k: (0, 0))
    vec = pl.BlockSpec((1, 128), lambda k: (0, 0))
    out_sh = jax.ShapeDtypeStruct((NM_PAD, 128), jnp.float32)
    return pl.pallas_call(
        _ea_body,
        grid=(NM_PAD // _MT,),
        in_specs=[row(0, 0), row(0, 0), row(0, 0), full, full, vec, full,
                  vec, vec, vec, full, full],
        out_specs=[row(0, 0), row(0, 0), row(0, 0)],
        out_shape=[out_sh, out_sh, out_sh],
    )(x, a0, a1, wn1[:ND], wn1[ND:], pn['l1']['b'][None], pn['l2']['w'],
      pn['l2']['b'][None], pn['g'][None], pn['be'][None], wa_next, wb_next)


def _c_body(ga_ref, gb_ref, e_ref, wc_ref, b1_ref, w2_ref, b2_ref, g_ref,
            be_ref, out_ref):
    e = e_ref[...]
    ec = jnp.dot(e, wc_ref[...], preferred_element_type=jnp.float32) + b1_ref[...]
    h1 = jnp.maximum(ga_ref[...] + gb_ref[...] + ec, 0.0)
    h2 = jnp.dot(h1, w2_ref[...], preferred_element_type=jnp.float32) + b2_ref[...]
    out_ref[...] = e + _ln(h2, g_ref[...], be_ref[...])


def _c_call(ga, gb, e, pe):
    w1 = pe['l1']['w']
    row = pl.BlockSpec((_ET, 128), lambda k: (k, 0))
    full = pl.BlockSpec((128, 128), lambda k: (0, 0))
    vec = pl.BlockSpec((1, 128), lambda k: (0, 0))
    return pl.pallas_call(
        _c_body,
        grid=(EP_PAD // _ET,),
        in_specs=[row, row, row, full, vec, full, vec, vec, vec],
        out_specs=row,
        out_shape=jax.ShapeDtypeStruct((EP_PAD, 128), jnp.float32),
    )(ga, gb, e, w1[2 * ND:], pe['l1']['b'][None], pe['l2']['w'],
      pe['l2']['b'][None], pe['g'][None], pe['be'][None])


def _ef_body(f_ref, w1_ref, b1_ref, w2_ref, b2_ref, g_ref, be_ref, out_ref):
    h = jnp.maximum(
        jnp.dot(f_ref[...], w1_ref[...], preferred_element_type=jnp.float32)
        + b1_ref[...], 0.0)
    h2 = jnp.dot(h, w2_ref[...], preferred_element_type=jnp.float32) + b2_ref[...]
    out_ref[...] = _ln(h2, g_ref[...], be_ref[...])


def _ef_call(pf_pad, p):
    w1 = jnp.zeros((8, 128), jnp.float32).at[:3].set(p['l1']['w'])
    row_in = pl.BlockSpec((_ET, 8), lambda k: (k, 0))
    row_out = pl.BlockSpec((_ET, 128), lambda k: (k, 0))
    vec = pl.BlockSpec((1, 128), lambda k: (0, 0))
    return pl.pallas_call(
        _ef_body,
        grid=(EP_PAD // _ET,),
        in_specs=[row_in, pl.BlockSpec((8, 128), lambda k: (0, 0)), vec,
                  pl.BlockSpec((128, 128), lambda k: (0, 0)), vec, vec, vec],
        out_specs=row_out,
        out_shape=jax.ShapeDtypeStruct((EP_PAD, 128), jnp.float32),
    )(pf_pad, w1, p['l1']['b'][None], p['l2']['w'], p['l2']['b'][None],
      p['g'][None], p['be'][None])


def _enc1_body(f_ref, ef_ref,
               nw1_ref, nb1_ref, nw2_ref, nb2_ref, ng_ref, nbe_ref,
               ew1_ref, eb1_ref, ew2_ref, eb2_ref, eg_ref, ebe_ref,
               wa_ref, wc_ref, gb1_ref, gw2_ref, gb2_ref, gg_ref, gbe_ref,
               xg_ref, m_ref):
    h = jnp.maximum(
        jnp.dot(f_ref[...], nw1_ref[...], preferred_element_type=jnp.float32)
        + nb1_ref[...], 0.0)
    xg = _ln(jnp.dot(h, nw2_ref[...], preferred_element_type=jnp.float32)
             + nb2_ref[...], ng_ref[...], nbe_ref[...])
    xg_ref[...] = xg
    h = jnp.maximum(
        jnp.dot(ef_ref[...], ew1_ref[...], preferred_element_type=jnp.float32)
        + eb1_ref[...], 0.0)
    ee = _ln(jnp.dot(h, ew2_ref[...], preferred_element_type=jnp.float32)
             + eb2_ref[...], eg_ref[...], ebe_ref[...])
    h = jnp.maximum(
        jnp.dot(xg, wa_ref[...], preferred_element_type=jnp.float32)
        + jnp.dot(ee, wc_ref[...], preferred_element_type=jnp.float32)
        + gb1_ref[...], 0.0)
    m_ref[...] = _ln(jnp.dot(h, gw2_ref[...], preferred_element_type=jnp.float32)
                     + gb2_ref[...], gg_ref[...], gbe_ref[...])


def _enc1_call(feats_p, enc_ef_p, params):
    pn, pe, pg = params['enc_node'], params['enc_edge'], params['enc_gn_e']
    nw1 = jnp.zeros((80, 128), jnp.float32).at[:FEAT].set(pn['l1']['w'])
    ew1 = jnp.zeros((8, 128), jnp.float32).at[:3].set(pe['l1']['w'])
    gw1 = pg['l1']['w']
    nb = pl.BlockSpec(None, lambda: (0, 0))
    out_sh = jax.ShapeDtypeStruct((N_GRID, 128), jnp.float32)
    return pl.pallas_call(
        _enc1_body,
        in_specs=[nb] * 21,
        out_specs=[nb, nb],
        out_shape=[out_sh, out_sh],
    )(feats_p, enc_ef_p,
      nw1, pn['l1']['b'][None], pn['l2']['w'], pn['l2']['b'][None],
      pn['g'][None], pn['be'][None],
      ew1, pe['l1']['b'][None], pe['l2']['w'], pe['l2']['b'][None],
      pe['g'][None], pe['be'][None],
      gw1[:ND], gw1[2 * ND:], pg['l1']['b'][None], pg['l2']['w'],
      pg['l2']['b'][None], pg['g'][None], pg['be'][None])


def _dec_body(xg_ref, gd_ref, ef_ref, f_ref,
              ew1_ref, eb1_ref, ew2_ref, eb2_ref, eg_ref, ebe_ref,
              wb_ref, wc_ref, gb1_ref, gw2_ref, gb2_ref, gg_ref, gbe_ref,
              wna_ref, wnb_ref, nb1_ref, nw2_ref, nb2_ref, ng_ref, nbe_ref,
              ow1_ref, ob1_ref, ow2_ref, ob2_ref,
              out_ref):
    xg = xg_ref[...]
    h = jnp.maximum(
        jnp.dot(ef_ref[...], ew1_ref[...], preferred_element_type=jnp.float32)
        + eb1_ref[...], 0.0)
    ed = _ln(jnp.dot(h, ew2_ref[...], preferred_element_type=jnp.float32)
             + eb2_ref[...], eg_ref[...], ebe_ref[...])
    h = jnp.maximum(
        gd_ref[...]
        + jnp.dot(xg, wb_ref[...], preferred_element_type=jnp.float32)
        + jnp.dot(ed, wc_ref[...], preferred_element_type=jnp.float32)
        + gb1_ref[...], 0.0)
    m = _ln(jnp.dot(h, gw2_ref[...], preferred_element_type=jnp.float32)
            + gb2_ref[...], gg_ref[...], gbe_ref[...])
    h = jnp.maximum(
        jnp.dot(xg, wna_ref[...], preferred_element_type=jnp.float32)
        + jnp.dot(m, wnb_ref[...], preferred_element_type=jnp.float32)
        + nb1_ref[...], 0.0)
    x_out = _ln(jnp.dot(h, nw2_ref[...], preferred_element_type=jnp.float32)
                + nb2_ref[...], ng_ref[...], nbe_ref[...])
    d1 = jnp.maximum(
        jnp.dot(x_out, ow1_ref[...], preferred_element_type=jnp.float32)
        + ob1_ref[...], 0.0)
    delta = jnp.dot(d1, ow2_ref[...], preferred_element_type=jnp.float32) + ob2_ref[...]
    out_ref[...] = f_ref[...] + delta


def _dec_call(x_grid, gdec, dec_ef_p, feats_p, params):
    pe, pg, pn, po = (params['dec_edge'], params['dec_gn_e'],
                      params['dec_gn_n'], params['dec_out'])
    ew1 = jnp.zeros((8, 128), jnp.float32).at[:3].set(pe['l1']['w'])
    gw1 = pg['l1']['w']
    nw1 = pn['l1']['w']
    ow2 = jnp.zeros((HDD, 80), jnp.float32).at[:, :FEAT].set(po['l2']['w'])
    ob2 = jnp.zeros((1, 80), jnp.float32).at[0, :FEAT].set(po['l2']['b'])
    nb = pl.BlockSpec(None, lambda: (0, 0))
    return pl.pallas_call(
        _dec_body,
        in_specs=[nb] * 28,
        out_specs=nb,
        out_shape=jax.ShapeDtypeStruct((N_GRID, 80), jnp.float32),
    )(x_grid, gdec, dec_ef_p, feats_p,
      ew1, pe['l1']['b'][None], pe['l2']['w'], pe['l2']['b'][None],
      pe['g'][None], pe['be'][None],
      gw1[ND:2 * ND], gw1[2 * ND:], pg['l1']['b'][None], pg['l2']['w'],
      pg['l2']['b'][None], pg['g'][None], pg['be'][None],
      nw1[:ND], nw1[ND:], pn['l1']['b'][None], pn['l2']['w'],
      pn['l2']['b'][None], pn['g'][None], pn['be'][None],
      po['l1']['w'], po['l1']['b'][None], ow2, ob2)


_NW = 32
_NMT = NM_PAD // 16


def _make_gather_pair(nchunks):
    epw = nchunks * 128
    e_tot = _NW * epw
    mesh = plsc.VectorSubcoreMesh(core_axis_name="c", subcore_axis_name="s")
    out_sh = jax.ShapeDtypeStruct((e_tot, 128), jnp.float32)

    @functools.partial(
        pl.kernel, out_type=[out_sh, out_sh], mesh=mesh,
        scratch_types=[
            pltpu.VMEM((nchunks, 128), jnp.int32),
            pltpu.VMEM((nchunks, 128), jnp.int32),
            pltpu.VMEM((128, 128), jnp.float32),
            pltpu.VMEM((128, 128), jnp.float32),
            pltpu.SemaphoreType.DMA,
            pltpu.SemaphoreType.DMA,
        ])
    def g(xa, xb, ps3, pd3, ga, gb, ia, ib, abuf, bbuf, sema, semb):
        wid = lax.axis_index("s") * 2 + lax.axis_index("c")
        base = wid * epw
        pltpu.sync_copy(ps3.at[wid], ia)
        pltpu.sync_copy(pd3.at[wid], ib)
        for j in range(nchunks):
            cpa = pltpu.async_copy(xa.at[ia.at[j]], abuf, sema)
            cpb = pltpu.async_copy(xb.at[ib.at[j]], bbuf, semb)
            cpa.wait()
            pltpu.sync_copy(abuf, ga.at[pl.ds(base + j * 128, 128)])
            cpb.wait()
            pltpu.sync_copy(bbuf, gb.at[pl.ds(base + j * 128, 128)])

    return g


def _make_gather_one(nchunks):
    epw = nchunks * 128
    e_tot = _NW * epw
    mesh = plsc.VectorSubcoreMesh(core_axis_name="c", subcore_axis_name="s")
    out_sh = jax.ShapeDtypeStruct((e_tot, 128), jnp.float32)

    @functools.partial(
        pl.kernel, out_type=out_sh, mesh=mesh,
        scratch_types=[
            pltpu.VMEM((nchunks, 128), jnp.int32),
            pltpu.VMEM((128, 128), jnp.float32),
            pltpu.SemaphoreType.DMA,
        ])
    def g(xa, ps3, ga, ia, abuf, sema):
        wid = lax.axis_index("s") * 2 + lax.axis_index("c")
        base = wid * epw
        pltpu.sync_copy(ps3.at[wid], ia)
        for j in range(nchunks):
            pltpu.async_copy(xa.at[ia.at[j]], abuf, sema).wait()
            pltpu.sync_copy(abuf, ga.at[pl.ds(base + j * 128, 128)])

    return g


def _make_scatter(nchunks):
    epw = nchunks * 128
    mesh = plsc.VectorSubcoreMesh(core_axis_name="c", subcore_axis_name="s")
    out_sh = jax.ShapeDtypeStruct((2, NM_PAD, 128), jnp.float32)

    @functools.partial(
        pl.kernel, out_type=out_sh, mesh=mesh,
        scratch_types=[
            pltpu.VMEM((nchunks, 128), jnp.int32),
            pltpu.VMEM((128, 128), jnp.float32),
            pltpu.VMEM_SHARED((NM_PAD, 128), jnp.float32),
        ])
    def s(e_hbm, idx3, zeros_hbm, out, ib, ebuf, acc):
        c = lax.axis_index("c")
        sid = lax.axis_index("s")
        wid = sid * 2 + c
        base = wid * epw
        pltpu.sync_copy(idx3.at[wid], ib)
        pltpu.sync_copy(zeros_hbm.at[pl.ds(sid * _NMT, _NMT)],
                        acc.at[pl.ds(sid * _NMT, _NMT)])
        plsc.subcore_barrier()
        for j in range(nchunks):
            pltpu.sync_copy(e_hbm.at[pl.ds(base + j * 128, 128)], ebuf)
            pltpu.sync_copy(ebuf, acc.at[ib.at[j]], add=True)
        plsc.subcore_barrier()
        pltpu.sync_copy(acc.at[pl.ds(sid * _NMT, _NMT)],
                        out.at[c, pl.ds(sid * _NMT, _NMT)])

    return s


_gather_proc = _make_gather_pair(9)
_gather_dec = _make_gather_one(1)
_scatter_proc = _make_scatter(9)
_scatter_enc = _make_scatter(1)


@jax.jit
def _run(features, params, enc_ef, proc_ef, dec_ef, enc_edges, proc_edges,
         dec_edges):
    feats_p = jnp.zeros((N_GRID, 80), jnp.float32).at[:, :FEAT].set(
        features.reshape(N_GRID, FEAT))
    enc_ef_p = jnp.zeros((N_GRID, 8), jnp.float32).at[:, :3].set(enc_ef)
    dec_ef_p = jnp.zeros((N_GRID, 8), jnp.float32).at[:, :3].set(dec_ef)
    pf_pad = jnp.zeros((EP_PAD, 8), jnp.float32).at[:E_PROC, :3].set(proc_ef)

    ps = proc_edges[0]
    pd = proc_edges[1]
    ps3 = jnp.pad(ps, (0, EP_PAD - E_PROC)).reshape(_NW, 9, 128)
    pd3 = jnp.pad(pd, (0, EP_PAD - E_PROC)).reshape(_NW, 9, 128)
    pds3 = jnp.pad(pd, (0, EP_PAD - E_PROC),
                   constant_values=N_MESH).reshape(_NW, 9, 128)
    encd3 = jnp.pad(enc_edges[1], (0, NG_PAD - N_GRID),
                    constant_values=N_MESH).reshape(_NW, 1, 128)
    ds3 = jnp.pad(dec_edges[0], (0, NG_PAD - N_GRID)).reshape(_NW, 1, 128)
    zeros_hbm = jnp.zeros((NM_PAD, 128), jnp.float32)

    x_grid, m = _enc1_call(feats_p, enc_ef_p, params)
    m_pad = jnp.zeros((NG_PAD, 128), jnp.float32).at[:N_GRID].set(m)
    agg = _scatter_enc(m_pad, encd3, zeros_hbm)
    blk0 = params['blocks'][0]
    w1n = blk0['e']['l1']['w']
    zeros_x = jnp.zeros((NM_PAD, 128), jnp.float32)
    x, xa, xb = _ea_call(zeros_x, agg[0], agg[1], params['enc_gn_n'],
                         w1n[:ND], w1n[ND:2 * ND])

    e = _ef_call(pf_pad, params['proc_edge'])

    nb = len(params['blocks'])
    gw1 = params['dec_gn_e']['l1']['w']
    for k, blk in enumerate(params['blocks']):
        ga, gb = _gather_proc(xa, xb, ps3, pd3)
        e = _c_call(ga, gb, e, blk['e'])
        agg = _scatter_proc(e, pds3, zeros_hbm)
        if k + 1 < nb:
            w1n = params['blocks'][k + 1]['e']['l1']['w']
            wa_next, wb_next = w1n[:ND], w1n[ND:2 * ND]
        else:
            wa_next, wb_next = gw1[:ND], gw1[ND:2 * ND]
        x, xa, xb = _ea_call(x, agg[0], agg[1], blk['n'], wa_next, wb_next)

    gdec = _gather_dec(xa, ds3)[:N_GRID]
    out = _dec_call(x_grid, gdec, dec_ef_p, feats_p, params)
    return out[:, :FEAT].reshape(1, N_GRID, FEAT)


def kernel(features, params, enc_ef, proc_ef, dec_ef, enc_edges, proc_edges,
           dec_edges):
    return _run(features, params, enc_ef, proc_ef, dec_ef, enc_edges,
                proc_edges, dec_edges)


# speedup vs baseline: 2.1014x; 1.0542x over previous
import functools
import jax
import jax.numpy as jnp
from jax import lax
from jax.experimental import pallas as pl
from jax.experimental.pallas import tpu as pltpu
from jax.experimental.pallas import tpu_sc as plsc

N_GRID = 648
N_MESH = 5882
E_PROC = N_MESH * 6
FEAT = 78
ND = 128
ED = 128
HDD = 64

NM_PAD = 5888
EP_PAD = 36864
NG_PAD = 4096

_MT = 736
_ET = 2304


def _ln(h, g, be):
    mu = jnp.mean(h, axis=-1, keepdims=True)
    v = jnp.mean((h - mu) * (h - mu), axis=-1, keepdims=True)
    return (h - mu) * lax.rsqrt(v + 1e-5) * g + be


def _ea_body(x_ref, a0_ref, a1_ref, wna_ref, wnb_ref, bn1_ref, wn2_ref,
             bn2_ref, g_ref, be_ref, wa_ref, wb_ref,
             xn_ref, xa_ref, xb_ref):
    agg = a0_ref[...] + a1_ref[...]
    x = x_ref[...]
    h = jnp.maximum(
        jnp.dot(x, wna_ref[...], preferred_element_type=jnp.float32)
        + jnp.dot(agg, wnb_ref[...], preferred_element_type=jnp.float32)
        + bn1_ref[...], 0.0)
    h2 = jnp.dot(h, wn2_ref[...], preferred_element_type=jnp.float32) + bn2_ref[...]
    xn = x + _ln(h2, g_ref[...], be_ref[...])
    xn_ref[...] = xn
    xa_ref[...] = jnp.dot(xn, wa_ref[...], preferred_element_type=jnp.float32)
    xb_ref[...] = jnp.dot(xn, wb_ref[...], preferred_element_type=jnp.float32)


def _ea_call(x, a0, a1, pn, wa_next, wb_next):
    wn1 = pn['l1']['w']
    row = lambda i, j: pl.BlockSpec((_MT, 128), lambda k: (k, 0))
    full = pl.BlockSpec((128, 128), lambda k: (0, 0))
    vec = pl.BlockSpec((1, 128), lambda k: (0, 0))
    out_sh = jax.ShapeDtypeStruct((NM_PAD, 128), jnp.float32)
    return pl.pallas_call(
        _ea_body,
        grid=(NM_PAD // _MT,),
        in_specs=[row(0, 0), row(0, 0), row(0, 0), full, full, vec, full,
                  vec, vec, vec, full, full],
        out_specs=[row(0, 0), row(0, 0), row(0, 0)],
        out_shape=[out_sh, out_sh, out_sh],
    )(x, a0, a1, wn1[:ND], wn1[ND:], pn['l1']['b'][None], pn['l2']['w'],
      pn['l2']['b'][None], pn['g'][None], pn['be'][None], wa_next, wb_next)


def _c_body(ga_ref, gb_ref, e_ref, wc_ref, b1_ref, w2_ref, b2_ref, g_ref,
            be_ref, out_ref):
    e = e_ref[...]
    ec = jnp.dot(e, wc_ref[...], preferred_element_type=jnp.float32) + b1_ref[...]
    h1 = jnp.maximum(ga_ref[...] + gb_ref[...] + ec, 0.0)
    h2 = jnp.dot(h1, w2_ref[...], preferred_element_type=jnp.float32) + b2_ref[...]
    out_ref[...] = e + _ln(h2, g_ref[...], be_ref[...])


def _c_call(ga, gb, e, pe):
    w1 = pe['l1']['w']
    row = pl.BlockSpec((_ET, 128), lambda k: (k, 0))
    full = pl.BlockSpec((128, 128), lambda k: (0, 0))
    vec = pl.BlockSpec((1, 128), lambda k: (0, 0))
    return pl.pallas_call(
        _c_body,
        grid=(EP_PAD // _ET,),
        in_specs=[row, row, row, full, vec, full, vec, vec, vec],
        out_specs=row,
        out_shape=jax.ShapeDtypeStruct((EP_PAD, 128), jnp.float32),
    )(ga, gb, e, w1[2 * ND:], pe['l1']['b'][None], pe['l2']['w'],
      pe['l2']['b'][None], pe['g'][None], pe['be'][None])


def _ef_body(f_ref, w1_ref, b1_ref, w2_ref, b2_ref, g_ref, be_ref, out_ref):
    h = jnp.maximum(
        jnp.dot(f_ref[...], w1_ref[...], preferred_element_type=jnp.float32)
        + b1_ref[...], 0.0)
    h2 = jnp.dot(h, w2_ref[...], preferred_element_type=jnp.float32) + b2_ref[...]
    out_ref[...] = _ln(h2, g_ref[...], be_ref[...])


def _ef_call(pf_pad, p):
    w1 = jnp.zeros((8, 128), jnp.float32).at[:3].set(p['l1']['w'])
    row_in = pl.BlockSpec((_ET, 8), lambda k: (k, 0))
    row_out = pl.BlockSpec((_ET, 128), lambda k: (k, 0))
    vec = pl.BlockSpec((1, 128), lambda k: (0, 0))
    return pl.pallas_call(
        _ef_body,
        grid=(EP_PAD // _ET,),
        in_specs=[row_in, pl.BlockSpec((8, 128), lambda k: (0, 0)), vec,
                  pl.BlockSpec((128, 128), lambda k: (0, 0)), vec, vec, vec],
        out_specs=row_out,
        out_shape=jax.ShapeDtypeStruct((EP_PAD, 128), jnp.float32),
    )(pf_pad, w1, p['l1']['b'][None], p['l2']['w'], p['l2']['b'][None],
      p['g'][None], p['be'][None])


def _enc1_body(f_ref, ef_ref,
               nw1_ref, nb1_ref, nw2_ref, nb2_ref, ng_ref, nbe_ref,
               ew1_ref, eb1_ref, ew2_ref, eb2_ref, eg_ref, ebe_ref,
               wa_ref, wc_ref, gb1_ref, gw2_ref, gb2_ref, gg_ref, gbe_ref,
               xg_ref, m_ref):
    h = jnp.maximum(
        jnp.dot(f_ref[...], nw1_ref[...], preferred_element_type=jnp.float32)
        + nb1_ref[...], 0.0)
    xg = _ln(jnp.dot(h, nw2_ref[...], preferred_element_type=jnp.float32)
             + nb2_ref[...], ng_ref[...], nbe_ref[...])
    xg_ref[...] = xg
    h = jnp.maximum(
        jnp.dot(ef_ref[...], ew1_ref[...], preferred_element_type=jnp.float32)
        + eb1_ref[...], 0.0)
    ee = _ln(jnp.dot(h, ew2_ref[...], preferred_element_type=jnp.float32)
             + eb2_ref[...], eg_ref[...], ebe_ref[...])
    h = jnp.maximum(
        jnp.dot(xg, wa_ref[...], preferred_element_type=jnp.float32)
        + jnp.dot(ee, wc_ref[...], preferred_element_type=jnp.float32)
        + gb1_ref[...], 0.0)
    m_ref[...] = _ln(jnp.dot(h, gw2_ref[...], preferred_element_type=jnp.float32)
                     + gb2_ref[...], gg_ref[...], gbe_ref[...])


def _enc1_call(feats_p, enc_ef_p, params):
    pn, pe, pg = params['enc_node'], params['enc_edge'], params['enc_gn_e']
    nw1 = jnp.zeros((80, 128), jnp.float32).at[:FEAT].set(pn['l1']['w'])
    ew1 = jnp.zeros((8, 128), jnp.float32).at[:3].set(pe['l1']['w'])
    gw1 = pg['l1']['w']
    nb = pl.BlockSpec(None, lambda: (0, 0))
    out_sh = jax.ShapeDtypeStruct((N_GRID, 128), jnp.float32)
    return pl.pallas_call(
        _enc1_body,
        in_specs=[nb] * 21,
        out_specs=[nb, nb],
        out_shape=[out_sh, out_sh],
    )(feats_p, enc_ef_p,
      nw1, pn['l1']['b'][None], pn['l2']['w'], pn['l2']['b'][None],
      pn['g'][None], pn['be'][None],
      ew1, pe['l1']['b'][None], pe['l2']['w'], pe['l2']['b'][None],
      pe['g'][None], pe['be'][None],
      gw1[:ND], gw1[2 * ND:], pg['l1']['b'][None], pg['l2']['w'],
      pg['l2']['b'][None], pg['g'][None], pg['be'][None])


def _dec_body(xg_ref, gd_ref, ef_ref, f_ref,
              ew1_ref, eb1_ref, ew2_ref, eb2_ref, eg_ref, ebe_ref,
              wb_ref, wc_ref, gb1_ref, gw2_ref, gb2_ref, gg_ref, gbe_ref,
              wna_ref, wnb_ref, nb1_ref, nw2_ref, nb2_ref, ng_ref, nbe_ref,
              ow1_ref, ob1_ref, ow2_ref, ob2_ref,
              out_ref):
    xg = xg_ref[...]
    h = jnp.maximum(
        jnp.dot(ef_ref[...], ew1_ref[...], preferred_element_type=jnp.float32)
        + eb1_ref[...], 0.0)
    ed = _ln(jnp.dot(h, ew2_ref[...], preferred_element_type=jnp.float32)
             + eb2_ref[...], eg_ref[...], ebe_ref[...])
    h = jnp.maximum(
        gd_ref[...]
        + jnp.dot(xg, wb_ref[...], preferred_element_type=jnp.float32)
        + jnp.dot(ed, wc_ref[...], preferred_element_type=jnp.float32)
        + gb1_ref[...], 0.0)
    m = _ln(jnp.dot(h, gw2_ref[...], preferred_element_type=jnp.float32)
            + gb2_ref[...], gg_ref[...], gbe_ref[...])
    h = jnp.maximum(
        jnp.dot(xg, wna_ref[...], preferred_element_type=jnp.float32)
        + jnp.dot(m, wnb_ref[...], preferred_element_type=jnp.float32)
        + nb1_ref[...], 0.0)
    x_out = _ln(jnp.dot(h, nw2_ref[...], preferred_element_type=jnp.float32)
                + nb2_ref[...], ng_ref[...], nbe_ref[...])
    d1 = jnp.maximum(
        jnp.dot(x_out, ow1_ref[...], preferred_element_type=jnp.float32)
        + ob1_ref[...], 0.0)
    delta = jnp.dot(d1, ow2_ref[...], preferred_element_type=jnp.float32) + ob2_ref[...]
    out_ref[...] = f_ref[...] + delta


def _dec_call(x_grid, gdec, dec_ef_p, feats_p, params):
    pe, pg, pn, po = (params['dec_edge'], params['dec_gn_e'],
                      params['dec_gn_n'], params['dec_out'])
    ew1 = jnp.zeros((8, 128), jnp.float32).at[:3].set(pe['l1']['w'])
    gw1 = pg['l1']['w']
    nw1 = pn['l1']['w']
    ow2 = jnp.zeros((HDD, 80), jnp.float32).at[:, :FEAT].set(po['l2']['w'])
    ob2 = jnp.zeros((1, 80), jnp.float32).at[0, :FEAT].set(po['l2']['b'])
    nb = pl.BlockSpec(None, lambda: (0, 0))
    return pl.pallas_call(
        _dec_body,
        in_specs=[nb] * 28,
        out_specs=nb,
        out_shape=jax.ShapeDtypeStruct((N_GRID, 80), jnp.float32),
    )(x_grid, gdec, dec_ef_p, feats_p,
      ew1, pe['l1']['b'][None], pe['l2']['w'], pe['l2']['b'][None],
      pe['g'][None], pe['be'][None],
      gw1[ND:2 * ND], gw1[2 * ND:], pg['l1']['b'][None], pg['l2']['w'],
      pg['l2']['b'][None], pg['g'][None], pg['be'][None],
      nw1[:ND], nw1[ND:], pn['l1']['b'][None], pn['l2']['w'],
      pn['l2']['b'][None], pn['g'][None], pn['be'][None],
      po['l1']['w'], po['l1']['b'][None], ow2, ob2)


_NW = 32
_NMT = NM_PAD // 16


def _make_gather_pair(nchunks):
    epw = nchunks * 128
    e_tot = _NW * epw
    mesh = plsc.VectorSubcoreMesh(core_axis_name="c", subcore_axis_name="s")
    out_sh = jax.ShapeDtypeStruct((e_tot, 128), jnp.float32)

    @functools.partial(
        pl.kernel, out_type=[out_sh, out_sh], mesh=mesh,
        scratch_types=[
            pltpu.VMEM((nchunks, 128), jnp.int32),
            pltpu.VMEM((nchunks, 128), jnp.int32),
            pltpu.VMEM((128, 128), jnp.float32),
            pltpu.VMEM((128, 128), jnp.float32),
            pltpu.VMEM((128, 128), jnp.float32),
            pltpu.VMEM((128, 128), jnp.float32),
            pltpu.SemaphoreType.DMA,
            pltpu.SemaphoreType.DMA,
            pltpu.SemaphoreType.DMA,
            pltpu.SemaphoreType.DMA,
            pltpu.SemaphoreType.DMA,
            pltpu.SemaphoreType.DMA,
        ])
    def g(xa, xb, ps3, pd3, ga, gb, ia, ib, a0, a1, b0, b1,
          sga, sgb, swa0, swa1, swb0, swb1):
        wid = lax.axis_index("s") * 2 + lax.axis_index("c")
        base = wid * epw
        pltpu.sync_copy(ps3.at[wid], ia)
        pltpu.sync_copy(pd3.at[wid], ib)
        ab = [a0, a1]
        bb = [b0, b1]
        swa = [swa0, swa1]
        swb = [swb0, swb1]
        gath = [None] * nchunks
        wb = [None] * nchunks
        gath[0] = (pltpu.async_copy(xa.at[ia.at[0]], a0, sga),
                   pltpu.async_copy(xb.at[ib.at[0]], b0, sgb))
        for j in range(nchunks):
            p = j % 2
            ca, cb = gath[j]
            ca.wait()
            cb.wait()
            wb[j] = (
                pltpu.async_copy(ab[p], ga.at[pl.ds(base + j * 128, 128)], swa[p]),
                pltpu.async_copy(bb[p], gb.at[pl.ds(base + j * 128, 128)], swb[p]))
            if j + 1 < nchunks:
                if j >= 1:
                    wa, wbk = wb[j - 1]
                    wa.wait()
                    wbk.wait()
                gath[j + 1] = (
                    pltpu.async_copy(xa.at[ia.at[j + 1]], ab[1 - p], sga),
                    pltpu.async_copy(xb.at[ib.at[j + 1]], bb[1 - p], sgb))
        for j in (nchunks - 2, nchunks - 1):
            if j >= 0 and wb[j] is not None:
                wa, wbk = wb[j]
                wa.wait()
                wbk.wait()

    return g


def _make_gather_one(nchunks):
    epw = nchunks * 128
    e_tot = _NW * epw
    mesh = plsc.VectorSubcoreMesh(core_axis_name="c", subcore_axis_name="s")
    out_sh = jax.ShapeDtypeStruct((e_tot, 128), jnp.float32)

    @functools.partial(
        pl.kernel, out_type=out_sh, mesh=mesh,
        scratch_types=[
            pltpu.VMEM((nchunks, 128), jnp.int32),
            pltpu.VMEM((128, 128), jnp.float32),
            pltpu.SemaphoreType.DMA,
        ])
    def g(xa, ps3, ga, ia, abuf, sema):
        wid = lax.axis_index("s") * 2 + lax.axis_index("c")
        base = wid * epw
        pltpu.sync_copy(ps3.at[wid], ia)
        for j in range(nchunks):
            pltpu.async_copy(xa.at[ia.at[j]], abuf, sema).wait()
            pltpu.sync_copy(abuf, ga.at[pl.ds(base + j * 128, 128)])

    return g


def _make_scatter(nchunks):
    epw = nchunks * 128
    mesh = plsc.VectorSubcoreMesh(core_axis_name="c", subcore_axis_name="s")
    out_sh = jax.ShapeDtypeStruct((2, NM_PAD, 128), jnp.float32)

    @functools.partial(
        pl.kernel, out_type=out_sh, mesh=mesh,
        scratch_types=[
            pltpu.VMEM((nchunks, 128), jnp.int32),
            pltpu.VMEM((128, 128), jnp.float32),
            pltpu.VMEM((128, 128), jnp.float32),
            pltpu.VMEM_SHARED((NM_PAD, 128), jnp.float32),
            pltpu.SemaphoreType.DMA,
            pltpu.SemaphoreType.DMA,
        ])
    def s(e_hbm, idx3, zeros_hbm, out, ib, e0, e1, acc, sld, ssc):
        c = lax.axis_index("c")
        sid = lax.axis_index("s")
        wid = sid * 2 + c
        base = wid * epw
        cz = pltpu.async_copy(zeros_hbm.at[pl.ds(sid * _NMT, _NMT)],
                              acc.at[pl.ds(sid * _NMT, _NMT)], ssc)
        pltpu.sync_copy(idx3.at[wid], ib)
        eb = [e0, e1]
        ld = [None] * nchunks
        sc = [None] * nchunks
        ld[0] = pltpu.async_copy(e_hbm.at[pl.ds(base, 128)], e0, sld)
        cz.wait()
        plsc.subcore_barrier()
        for j in range(nchunks):
            p = j % 2
            ld[j].wait()
            if j + 1 < nchunks:
                if j >= 1:
                    sc[j - 1].wait()
                ld[j + 1] = pltpu.async_copy(
                    e_hbm.at[pl.ds(base + (j + 1) * 128, 128)], eb[1 - p], sld)
            sc[j] = pltpu.async_copy(eb[p], acc.at[ib.at[j]], ssc, add=True)
        for j in (nchunks - 2, nchunks - 1):
            if j >= 0 and sc[j] is not None:
                sc[j].wait()
        plsc.subcore_barrier()
        pltpu.sync_copy(acc.at[pl.ds(sid * _NMT, _NMT)],
                        out.at[c, pl.ds(sid * _NMT, _NMT)])

    return s


_gather_proc = _make_gather_pair(9)
_gather_dec = _make_gather_one(1)
_scatter_proc = _make_scatter(9)
_scatter_enc = _make_scatter(1)


@jax.jit
def _run(features, params, enc_ef, proc_ef, dec_ef, enc_edges, proc_edges,
         dec_edges):
    feats_p = jnp.zeros((N_GRID, 80), jnp.float32).at[:, :FEAT].set(
        features.reshape(N_GRID, FEAT))
    enc_ef_p = jnp.zeros((N_GRID, 8), jnp.float32).at[:, :3].set(enc_ef)
    dec_ef_p = jnp.zeros((N_GRID, 8), jnp.float32).at[:, :3].set(dec_ef)
    pf_pad = jnp.zeros((EP_PAD, 8), jnp.float32).at[:E_PROC, :3].set(proc_ef)

    ps = proc_edges[0]
    pd = proc_edges[1]
    ps3 = jnp.pad(ps, (0, EP_PAD - E_PROC)).reshape(_NW, 9, 128)
    pd3 = jnp.pad(pd, (0, EP_PAD - E_PROC)).reshape(_NW, 9, 128)
    pds3 = jnp.pad(pd, (0, EP_PAD - E_PROC),
                   constant_values=N_MESH).reshape(_NW, 9, 128)
    encd3 = jnp.pad(enc_edges[1], (0, NG_PAD - N_GRID),
                    constant_values=N_MESH).reshape(_NW, 1, 128)
    ds3 = jnp.pad(dec_edges[0], (0, NG_PAD - N_GRID)).reshape(_NW, 1, 128)
    zeros_hbm = jnp.zeros((NM_PAD, 128), jnp.float32)

    x_grid, m = _enc1_call(feats_p, enc_ef_p, params)
    m_pad = jnp.zeros((NG_PAD, 128), jnp.float32).at[:N_GRID].set(m)
    agg = _scatter_enc(m_pad, encd3, zeros_hbm)
    blk0 = params['blocks'][0]
    w1n = blk0['e']['l1']['w']
    zeros_x = jnp.zeros((NM_PAD, 128), jnp.float32)
    x, xa, xb = _ea_call(zeros_x, agg[0], agg[1], params['enc_gn_n'],
                         w1n[:ND], w1n[ND:2 * ND])

    e = _ef_call(pf_pad, params['proc_edge'])

    nb = len(params['blocks'])
    gw1 = params['dec_gn_e']['l1']['w']
    for k, blk in enumerate(params['blocks']):
        ga, gb = _gather_proc(xa, xb, ps3, pd3)
        e = _c_call(ga, gb, e, blk['e'])
        agg = _scatter_proc(e, pds3, zeros_hbm)
        if k + 1 < nb:
            w1n = params['blocks'][k + 1]['e']['l1']['w']
            wa_next, wb_next = w1n[:ND], w1n[ND:2 * ND]
        else:
            wa_next, wb_next = gw1[:ND], gw1[ND:2 * ND]
        x, xa, xb = _ea_call(x, agg[0], agg[1], blk['n'], wa_next, wb_next)

    gdec = _gather_dec(xa, ds3)[:N_GRID]
    out = _dec_call(x_grid, gdec, dec_ef_p, feats_p, params)
    return out[:, :FEAT].reshape(1, N_GRID, FEAT)


def kernel(features, params, enc_ef, proc_ef, dec_ef, enc_edges, proc_edges,
           dec_edges):
    return _run(features, params, enc_ef, proc_ef, dec_ef, enc_edges,
                proc_edges, dec_edges)


# speedup vs baseline: 2.1556x; 1.0258x over previous
import functools
import jax
import jax.numpy as jnp
from jax import lax
from jax.experimental import pallas as pl
from jax.experimental.pallas import tpu as pltpu
from jax.experimental.pallas import tpu_sc as plsc

N_GRID = 648
N_MESH = 5882
E_PROC = N_MESH * 6
FEAT = 78
ND = 128
ED = 128
HDD = 64

NM_PAD = 5888
EP_PAD = 36864
NG_PAD = 4096

_MT = 736
_ET = 2304


def _ln(h, g, be):
    mu = jnp.mean(h, axis=-1, keepdims=True)
    v = jnp.mean((h - mu) * (h - mu), axis=-1, keepdims=True)
    return (h - mu) * lax.rsqrt(v + 1e-5) * g + be


def _ea_body(x_ref, a0_ref, a1_ref, wna_ref, wnb_ref, bn1_ref, wn2_ref,
             bn2_ref, g_ref, be_ref, wa_ref, wb_ref,
             xn_ref, xa_ref, xb_ref):
    agg = a0_ref[...] + a1_ref[...]
    x = x_ref[...]
    h = jnp.maximum(
        jnp.dot(x, wna_ref[...], preferred_element_type=jnp.float32)
        + jnp.dot(agg, wnb_ref[...], preferred_element_type=jnp.float32)
        + bn1_ref[...], 0.0)
    h2 = jnp.dot(h, wn2_ref[...], preferred_element_type=jnp.float32) + bn2_ref[...]
    xn = x + _ln(h2, g_ref[...], be_ref[...])
    xn_ref[...] = xn
    xa_ref[...] = jnp.dot(xn, wa_ref[...], preferred_element_type=jnp.float32)
    xb_ref[...] = jnp.dot(xn, wb_ref[...], preferred_element_type=jnp.float32)


def _ea_call(x, a0, a1, pn, wa_next, wb_next):
    wn1 = pn['l1']['w']
    row = lambda i, j: pl.BlockSpec((_MT, 128), lambda k: (k, 0))
    full = pl.BlockSpec((128, 128), lambda k: (0, 0))
    vec = pl.BlockSpec((1, 128), lambda k: (0, 0))
    out_sh = jax.ShapeDtypeStruct((NM_PAD, 128), jnp.float32)
    return pl.pallas_call(
        _ea_body,
        grid=(NM_PAD // _MT,),
        in_specs=[row(0, 0), row(0, 0), row(0, 0), full, full, vec, full,
                  vec, vec, vec, full, full],
        out_specs=[row(0, 0), row(0, 0), row(0, 0)],
        out_shape=[out_sh, out_sh, out_sh],
    )(x, a0, a1, wn1[:ND], wn1[ND:], pn['l1']['b'][None], pn['l2']['w'],
      pn['l2']['b'][None], pn['g'][None], pn['be'][None], wa_next, wb_next)


def _c_body(ga_ref, gb_ref, e_ref, wc_ref, b1_ref, w2_ref, b2_ref, g_ref,
            be_ref, out_ref):
    e = e_ref[...]
    ec = jnp.dot(e, wc_ref[...], preferred_element_type=jnp.float32) + b1_ref[...]
    h1 = jnp.maximum(ga_ref[...] + gb_ref[...] + ec, 0.0)
    h2 = jnp.dot(h1, w2_ref[...], preferred_element_type=jnp.float32) + b2_ref[...]
    out_ref[...] = e + _ln(h2, g_ref[...], be_ref[...])


def _c_call(ga, gb, e, pe):
    w1 = pe['l1']['w']
    row = pl.BlockSpec((_ET, 128), lambda k: (k, 0))
    full = pl.BlockSpec((128, 128), lambda k: (0, 0))
    vec = pl.BlockSpec((1, 128), lambda k: (0, 0))
    return pl.pallas_call(
        _c_body,
        grid=(EP_PAD // _ET,),
        in_specs=[row, row, row, full, vec, full, vec, vec, vec],
        out_specs=row,
        out_shape=jax.ShapeDtypeStruct((EP_PAD, 128), jnp.float32),
    )(ga, gb, e, w1[2 * ND:], pe['l1']['b'][None], pe['l2']['w'],
      pe['l2']['b'][None], pe['g'][None], pe['be'][None])


def _ef_body(f_ref, w1_ref, b1_ref, w2_ref, b2_ref, g_ref, be_ref, out_ref):
    h = jnp.maximum(
        jnp.dot(f_ref[...], w1_ref[...], preferred_element_type=jnp.float32)
        + b1_ref[...], 0.0)
    h2 = jnp.dot(h, w2_ref[...], preferred_element_type=jnp.float32) + b2_ref[...]
    out_ref[...] = _ln(h2, g_ref[...], be_ref[...])


def _ef_call(pf_pad, p):
    w1 = jnp.zeros((8, 128), jnp.float32).at[:3].set(p['l1']['w'])
    row_in = pl.BlockSpec((_ET, 8), lambda k: (k, 0))
    row_out = pl.BlockSpec((_ET, 128), lambda k: (k, 0))
    vec = pl.BlockSpec((1, 128), lambda k: (0, 0))
    return pl.pallas_call(
        _ef_body,
        grid=(EP_PAD // _ET,),
        in_specs=[row_in, pl.BlockSpec((8, 128), lambda k: (0, 0)), vec,
                  pl.BlockSpec((128, 128), lambda k: (0, 0)), vec, vec, vec],
        out_specs=row_out,
        out_shape=jax.ShapeDtypeStruct((EP_PAD, 128), jnp.float32),
    )(pf_pad, w1, p['l1']['b'][None], p['l2']['w'], p['l2']['b'][None],
      p['g'][None], p['be'][None])


def _enc1_body(f_ref, ef_ref,
               nw1_ref, nb1_ref, nw2_ref, nb2_ref, ng_ref, nbe_ref,
               ew1_ref, eb1_ref, ew2_ref, eb2_ref, eg_ref, ebe_ref,
               wa_ref, wc_ref, gb1_ref, gw2_ref, gb2_ref, gg_ref, gbe_ref,
               xg_ref, m_ref):
    h = jnp.maximum(
        jnp.dot(f_ref[...], nw1_ref[...], preferred_element_type=jnp.float32)
        + nb1_ref[...], 0.0)
    xg = _ln(jnp.dot(h, nw2_ref[...], preferred_element_type=jnp.float32)
             + nb2_ref[...], ng_ref[...], nbe_ref[...])
    xg_ref[...] = xg
    h = jnp.maximum(
        jnp.dot(ef_ref[...], ew1_ref[...], preferred_element_type=jnp.float32)
        + eb1_ref[...], 0.0)
    ee = _ln(jnp.dot(h, ew2_ref[...], preferred_element_type=jnp.float32)
             + eb2_ref[...], eg_ref[...], ebe_ref[...])
    h = jnp.maximum(
        jnp.dot(xg, wa_ref[...], preferred_element_type=jnp.float32)
        + jnp.dot(ee, wc_ref[...], preferred_element_type=jnp.float32)
        + gb1_ref[...], 0.0)
    m_ref[...] = _ln(jnp.dot(h, gw2_ref[...], preferred_element_type=jnp.float32)
                     + gb2_ref[...], gg_ref[...], gbe_ref[...])


def _enc1_call(feats_p, enc_ef_p, params):
    pn, pe, pg = params['enc_node'], params['enc_edge'], params['enc_gn_e']
    nw1 = jnp.zeros((80, 128), jnp.float32).at[:FEAT].set(pn['l1']['w'])
    ew1 = jnp.zeros((8, 128), jnp.float32).at[:3].set(pe['l1']['w'])
    gw1 = pg['l1']['w']
    nb = pl.BlockSpec(None, lambda: (0, 0))
    out_sh = jax.ShapeDtypeStruct((N_GRID, 128), jnp.float32)
    return pl.pallas_call(
        _enc1_body,
        in_specs=[nb] * 21,
        out_specs=[nb, nb],
        out_shape=[out_sh, out_sh],
    )(feats_p, enc_ef_p,
      nw1, pn['l1']['b'][None], pn['l2']['w'], pn['l2']['b'][None],
      pn['g'][None], pn['be'][None],
      ew1, pe['l1']['b'][None], pe['l2']['w'], pe['l2']['b'][None],
      pe['g'][None], pe['be'][None],
      gw1[:ND], gw1[2 * ND:], pg['l1']['b'][None], pg['l2']['w'],
      pg['l2']['b'][None], pg['g'][None], pg['be'][None])


def _dec_body(xg_ref, gd_ref, ef_ref, f_ref,
              ew1_ref, eb1_ref, ew2_ref, eb2_ref, eg_ref, ebe_ref,
              wb_ref, wc_ref, gb1_ref, gw2_ref, gb2_ref, gg_ref, gbe_ref,
              wna_ref, wnb_ref, nb1_ref, nw2_ref, nb2_ref, ng_ref, nbe_ref,
              ow1_ref, ob1_ref, ow2_ref, ob2_ref,
              out_ref):
    xg = xg_ref[...]
    h = jnp.maximum(
        jnp.dot(ef_ref[...], ew1_ref[...], preferred_element_type=jnp.float32)
        + eb1_ref[...], 0.0)
    ed = _ln(jnp.dot(h, ew2_ref[...], preferred_element_type=jnp.float32)
             + eb2_ref[...], eg_ref[...], ebe_ref[...])
    h = jnp.maximum(
        gd_ref[...]
        + jnp.dot(xg, wb_ref[...], preferred_element_type=jnp.float32)
        + jnp.dot(ed, wc_ref[...], preferred_element_type=jnp.float32)
        + gb1_ref[...], 0.0)
    m = _ln(jnp.dot(h, gw2_ref[...], preferred_element_type=jnp.float32)
            + gb2_ref[...], gg_ref[...], gbe_ref[...])
    h = jnp.maximum(
        jnp.dot(xg, wna_ref[...], preferred_element_type=jnp.float32)
        + jnp.dot(m, wnb_ref[...], preferred_element_type=jnp.float32)
        + nb1_ref[...], 0.0)
    x_out = _ln(jnp.dot(h, nw2_ref[...], preferred_element_type=jnp.float32)
                + nb2_ref[...], ng_ref[...], nbe_ref[...])
    d1 = jnp.maximum(
        jnp.dot(x_out, ow1_ref[...], preferred_element_type=jnp.float32)
        + ob1_ref[...], 0.0)
    delta = jnp.dot(d1, ow2_ref[...], preferred_element_type=jnp.float32) + ob2_ref[...]
    out_ref[...] = f_ref[...] + delta


def _dec_call(x_grid, gdec, dec_ef_p, feats_p, params):
    pe, pg, pn, po = (params['dec_edge'], params['dec_gn_e'],
                      params['dec_gn_n'], params['dec_out'])
    ew1 = jnp.zeros((8, 128), jnp.float32).at[:3].set(pe['l1']['w'])
    gw1 = pg['l1']['w']
    nw1 = pn['l1']['w']
    ow2 = jnp.zeros((HDD, 80), jnp.float32).at[:, :FEAT].set(po['l2']['w'])
    ob2 = jnp.zeros((1, 80), jnp.float32).at[0, :FEAT].set(po['l2']['b'])
    nb = pl.BlockSpec(None, lambda: (0, 0))
    return pl.pallas_call(
        _dec_body,
        in_specs=[nb] * 28,
        out_specs=nb,
        out_shape=jax.ShapeDtypeStruct((N_GRID, 80), jnp.float32),
    )(x_grid, gdec, dec_ef_p, feats_p,
      ew1, pe['l1']['b'][None], pe['l2']['w'], pe['l2']['b'][None],
      pe['g'][None], pe['be'][None],
      gw1[ND:2 * ND], gw1[2 * ND:], pg['l1']['b'][None], pg['l2']['w'],
      pg['l2']['b'][None], pg['g'][None], pg['be'][None],
      nw1[:ND], nw1[ND:], pn['l1']['b'][None], pn['l2']['w'],
      pn['l2']['b'][None], pn['g'][None], pn['be'][None],
      po['l1']['w'], po['l1']['b'][None], ow2, ob2)


_NW = 32
_NMT = NM_PAD // 16


def _make_gather_pair(nchunks):
    epw = nchunks * 128
    e_tot = _NW * epw
    mesh = plsc.VectorSubcoreMesh(core_axis_name="c", subcore_axis_name="s")
    out_sh = jax.ShapeDtypeStruct((e_tot, 128), jnp.float32)

    depth = min(3, nchunks)

    @functools.partial(
        pl.kernel, out_type=[out_sh, out_sh], mesh=mesh,
        scratch_types=[
            pltpu.VMEM((nchunks, 128), jnp.int32),
            pltpu.VMEM((nchunks, 128), jnp.int32),
            [pltpu.VMEM((128, 128), jnp.float32)] * depth,
            [pltpu.VMEM((128, 128), jnp.float32)] * depth,
            [pltpu.SemaphoreType.DMA] * depth,
            [pltpu.SemaphoreType.DMA] * depth,
            pltpu.SemaphoreType.DMA,
            pltpu.SemaphoreType.DMA,
        ])
    def g(xa, xb, ps3, pd3, ga, gb, ia, ib, ab, bb, sga, sgb, swa, swb):
        wid = lax.axis_index("s") * 2 + lax.axis_index("c")
        base = wid * epw
        pltpu.sync_copy(ps3.at[wid], ia)
        pltpu.sync_copy(pd3.at[wid], ib)
        gath = [None] * nchunks
        for j in range(depth):
            gath[j] = (pltpu.async_copy(xa.at[ia.at[j]], ab[j], sga[j]),
                       pltpu.async_copy(xb.at[ib.at[j]], bb[j], sgb[j]))
        for j in range(nchunks):
            p = j % depth
            ca, cb = gath[j]
            ca.wait()
            cb.wait()
            wa = pltpu.async_copy(ab[p], ga.at[pl.ds(base + j * 128, 128)], swa)
            wbk = pltpu.async_copy(bb[p], gb.at[pl.ds(base + j * 128, 128)], swb)
            if j + depth < nchunks:
                wa.wait()
                wbk.wait()
                gath[j + depth] = (
                    pltpu.async_copy(xa.at[ia.at[j + depth]], ab[p], sga[p]),
                    pltpu.async_copy(xb.at[ib.at[j + depth]], bb[p], sgb[p]))
            else:
                wa.wait()
                wbk.wait()

    return g


def _make_gather_one(nchunks):
    epw = nchunks * 128
    e_tot = _NW * epw
    mesh = plsc.VectorSubcoreMesh(core_axis_name="c", subcore_axis_name="s")
    out_sh = jax.ShapeDtypeStruct((e_tot, 128), jnp.float32)

    @functools.partial(
        pl.kernel, out_type=out_sh, mesh=mesh,
        scratch_types=[
            pltpu.VMEM((nchunks, 128), jnp.int32),
            pltpu.VMEM((128, 128), jnp.float32),
            pltpu.SemaphoreType.DMA,
        ])
    def g(xa, ps3, ga, ia, abuf, sema):
        wid = lax.axis_index("s") * 2 + lax.axis_index("c")
        base = wid * epw
        pltpu.sync_copy(ps3.at[wid], ia)
        for j in range(nchunks):
            pltpu.async_copy(xa.at[ia.at[j]], abuf, sema).wait()
            pltpu.sync_copy(abuf, ga.at[pl.ds(base + j * 128, 128)])

    return g


def _make_scatter(nchunks):
    epw = nchunks * 128
    mesh = plsc.VectorSubcoreMesh(core_axis_name="c", subcore_axis_name="s")
    out_sh = jax.ShapeDtypeStruct((2, NM_PAD, 128), jnp.float32)

    @functools.partial(
        pl.kernel, out_type=out_sh, mesh=mesh,
        scratch_types=[
            pltpu.VMEM((nchunks, 128), jnp.int32),
            pltpu.VMEM((128, 128), jnp.float32),
            pltpu.VMEM((128, 128), jnp.float32),
            pltpu.VMEM_SHARED((NM_PAD, 128), jnp.float32),
            pltpu.SemaphoreType.DMA,
            pltpu.SemaphoreType.DMA,
        ])
    def s(e_hbm, idx3, zeros_hbm, out, ib, e0, e1, acc, sld, ssc):
        c = lax.axis_index("c")
        sid = lax.axis_index("s")
        wid = sid * 2 + c
        base = wid * epw
        cz = pltpu.async_copy(zeros_hbm.at[pl.ds(sid * _NMT, _NMT)],
                              acc.at[pl.ds(sid * _NMT, _NMT)], ssc)
        pltpu.sync_copy(idx3.at[wid], ib)
        eb = [e0, e1]
        ld = [None] * nchunks
        sc = [None] * nchunks
        ld[0] = pltpu.async_copy(e_hbm.at[pl.ds(base, 128)], e0, sld)
        cz.wait()
        plsc.subcore_barrier()
        for j in range(nchunks):
            p = j % 2
            ld[j].wait()
            if j + 1 < nchunks:
                if j >= 1:
                    sc[j - 1].wait()
                ld[j + 1] = pltpu.async_copy(
                    e_hbm.at[pl.ds(base + (j + 1) * 128, 128)], eb[1 - p], sld)
            sc[j] = pltpu.async_copy(eb[p], acc.at[ib.at[j]], ssc, add=True)
        for j in (nchunks - 2, nchunks - 1):
            if j >= 0 and sc[j] is not None:
                sc[j].wait()
        plsc.subcore_barrier()
        pltpu.sync_copy(acc.at[pl.ds(sid * _NMT, _NMT)],
                        out.at[c, pl.ds(sid * _NMT, _NMT)])

    return s


_gather_proc = _make_gather_pair(9)
_gather_dec = _make_gather_one(1)
_scatter_proc = _make_scatter(9)
_scatter_enc = _make_scatter(1)


@jax.jit
def _run(features, params, enc_ef, proc_ef, dec_ef, enc_edges, proc_edges,
         dec_edges):
    feats_p = jnp.zeros((N_GRID, 80), jnp.float32).at[:, :FEAT].set(
        features.reshape(N_GRID, FEAT))
    enc_ef_p = jnp.zeros((N_GRID, 8), jnp.float32).at[:, :3].set(enc_ef)
    dec_ef_p = jnp.zeros((N_GRID, 8), jnp.float32).at[:, :3].set(dec_ef)
    pf_pad = jnp.zeros((EP_PAD, 8), jnp.float32).at[:E_PROC, :3].set(proc_ef)

    ps = proc_edges[0]
    pd = proc_edges[1]
    ps3 = jnp.pad(ps, (0, EP_PAD - E_PROC)).reshape(_NW, 9, 128)
    pd3 = jnp.pad(pd, (0, EP_PAD - E_PROC)).reshape(_NW, 9, 128)
    pds3 = jnp.pad(pd, (0, EP_PAD - E_PROC),
                   constant_values=N_MESH).reshape(_NW, 9, 128)
    encd3 = jnp.pad(enc_edges[1], (0, NG_PAD - N_GRID),
                    constant_values=N_MESH).reshape(_NW, 1, 128)
    ds3 = jnp.pad(dec_edges[0], (0, NG_PAD - N_GRID)).reshape(_NW, 1, 128)
    zeros_hbm = jnp.zeros((NM_PAD, 128), jnp.float32)

    x_grid, m = _enc1_call(feats_p, enc_ef_p, params)
    m_pad = jnp.zeros((NG_PAD, 128), jnp.float32).at[:N_GRID].set(m)
    agg = _scatter_enc(m_pad, encd3, zeros_hbm)
    blk0 = params['blocks'][0]
    w1n = blk0['e']['l1']['w']
    zeros_x = jnp.zeros((NM_PAD, 128), jnp.float32)
    x, xa, xb = _ea_call(zeros_x, agg[0], agg[1], params['enc_gn_n'],
                         w1n[:ND], w1n[ND:2 * ND])

    e = _ef_call(pf_pad, params['proc_edge'])

    nb = len(params['blocks'])
    gw1 = params['dec_gn_e']['l1']['w']
    for k, blk in enumerate(params['blocks']):
        ga, gb = _gather_proc(xa, xb, ps3, pd3)
        e = _c_call(ga, gb, e, blk['e'])
        agg = _scatter_proc(e, pds3, zeros_hbm)
        if k + 1 < nb:
            w1n = params['blocks'][k + 1]['e']['l1']['w']
            wa_next, wb_next = w1n[:ND], w1n[ND:2 * ND]
        else:
            wa_next, wb_next = gw1[:ND], gw1[ND:2 * ND]
        x, xa, xb = _ea_call(x, agg[0], agg[1], blk['n'], wa_next, wb_next)

    gdec = _gather_dec(xa, ds3)[:N_GRID]
    out = _dec_call(x_grid, gdec, dec_ef_p, feats_p, params)
    return out[:, :FEAT].reshape(1, N_GRID, FEAT)


def kernel(features, params, enc_ef, proc_ef, dec_ef, enc_edges, proc_edges,
           dec_edges):
    return _run(features, params, enc_ef, proc_ef, dec_ef, enc_edges,
                proc_edges, dec_edges)


# speedup vs baseline: 2.3478x; 1.0891x over previous
import functools
import jax
import jax.numpy as jnp
from jax import lax
from jax.experimental import pallas as pl
from jax.experimental.pallas import tpu as pltpu
from jax.experimental.pallas import tpu_sc as plsc

N_GRID = 648
N_MESH = 5882
E_PROC = N_MESH * 6
FEAT = 78
ND = 128
ED = 128
HDD = 64

NM_PAD = 5888
EP_PAD = 36864
NG_PAD = 4096

_MT = 736
_ET = 2304


def _ln(h, g, be):
    mu = jnp.mean(h, axis=-1, keepdims=True)
    v = jnp.mean((h - mu) * (h - mu), axis=-1, keepdims=True)
    return (h - mu) * lax.rsqrt(v + 1e-5) * g + be


def _ea_body(x_ref, a0_ref, a1_ref, wna_ref, wnb_ref, bn1_ref, wn2_ref,
             bn2_ref, g_ref, be_ref, wa_ref, wb_ref,
             xn_ref, xa_ref, xb_ref):
    agg = a0_ref[...] + a1_ref[...]
    x = x_ref[...]
    h = jnp.maximum(
        jnp.dot(x, wna_ref[...], preferred_element_type=jnp.float32)
        + jnp.dot(agg, wnb_ref[...], preferred_element_type=jnp.float32)
        + bn1_ref[...], 0.0)
    h2 = jnp.dot(h, wn2_ref[...], preferred_element_type=jnp.float32) + bn2_ref[...]
    xn = x + _ln(h2, g_ref[...], be_ref[...])
    xn_ref[...] = xn
    xa_ref[...] = jnp.dot(xn, wa_ref[...], preferred_element_type=jnp.float32)
    xb_ref[...] = jnp.dot(xn, wb_ref[...], preferred_element_type=jnp.float32)


def _ea_call(x, a0, a1, pn, wa_next, wb_next):
    wn1 = pn['l1']['w']
    row = lambda i, j: pl.BlockSpec((_MT, 128), lambda k: (k, 0))
    full = pl.BlockSpec((128, 128), lambda k: (0, 0))
    vec = pl.BlockSpec((1, 128), lambda k: (0, 0))
    out_sh = jax.ShapeDtypeStruct((NM_PAD, 128), jnp.float32)
    return pl.pallas_call(
        _ea_body,
        grid=(NM_PAD // _MT,),
        in_specs=[row(0, 0), row(0, 0), row(0, 0), full, full, vec, full,
                  vec, vec, vec, full, full],
        out_specs=[row(0, 0), row(0, 0), row(0, 0)],
        out_shape=[out_sh, out_sh, out_sh],
    )(x, a0, a1, wn1[:ND], wn1[ND:], pn['l1']['b'][None], pn['l2']['w'],
      pn['l2']['b'][None], pn['g'][None], pn['be'][None], wa_next, wb_next)


def _c_body(s_ref, e_ref, wc_ref, b1_ref, w2_ref, b2_ref, g_ref,
            be_ref, out_ref):
    e = e_ref[...]
    ec = jnp.dot(e, wc_ref[...], preferred_element_type=jnp.float32) + b1_ref[...]
    h1 = jnp.maximum(s_ref[...] + ec, 0.0)
    h2 = jnp.dot(h1, w2_ref[...], preferred_element_type=jnp.float32) + b2_ref[...]
    out_ref[...] = e + _ln(h2, g_ref[...], be_ref[...])


def _c_call(s, e, pe):
    w1 = pe['l1']['w']
    row = pl.BlockSpec((_ET, 128), lambda k: (k, 0))
    full = pl.BlockSpec((128, 128), lambda k: (0, 0))
    vec = pl.BlockSpec((1, 128), lambda k: (0, 0))
    return pl.pallas_call(
        _c_body,
        grid=(EP_PAD // _ET,),
        in_specs=[row, row, full, vec, full, vec, vec, vec],
        out_specs=row,
        out_shape=jax.ShapeDtypeStruct((EP_PAD, 128), jnp.float32),
    )(s, e, w1[2 * ND:], pe['l1']['b'][None], pe['l2']['w'],
      pe['l2']['b'][None], pe['g'][None], pe['be'][None])


def _ef_body(f_ref, w1_ref, b1_ref, w2_ref, b2_ref, g_ref, be_ref, out_ref):
    h = jnp.maximum(
        jnp.dot(f_ref[...], w1_ref[...], preferred_element_type=jnp.float32)
        + b1_ref[...], 0.0)
    h2 = jnp.dot(h, w2_ref[...], preferred_element_type=jnp.float32) + b2_ref[...]
    out_ref[...] = _ln(h2, g_ref[...], be_ref[...])


def _ef_call(pf_pad, p):
    w1 = jnp.zeros((8, 128), jnp.float32).at[:3].set(p['l1']['w'])
    row_in = pl.BlockSpec((_ET, 8), lambda k: (k, 0))
    row_out = pl.BlockSpec((_ET, 128), lambda k: (k, 0))
    vec = pl.BlockSpec((1, 128), lambda k: (0, 0))
    return pl.pallas_call(
        _ef_body,
        grid=(EP_PAD // _ET,),
        in_specs=[row_in, pl.BlockSpec((8, 128), lambda k: (0, 0)), vec,
                  pl.BlockSpec((128, 128), lambda k: (0, 0)), vec, vec, vec],
        out_specs=row_out,
        out_shape=jax.ShapeDtypeStruct((EP_PAD, 128), jnp.float32),
    )(pf_pad, w1, p['l1']['b'][None], p['l2']['w'], p['l2']['b'][None],
      p['g'][None], p['be'][None])


def _enc1_body(f_ref, ef_ref,
               nw1_ref, nb1_ref, nw2_ref, nb2_ref, ng_ref, nbe_ref,
               ew1_ref, eb1_ref, ew2_ref, eb2_ref, eg_ref, ebe_ref,
               wa_ref, wc_ref, gb1_ref, gw2_ref, gb2_ref, gg_ref, gbe_ref,
               xg_ref, m_ref):
    h = jnp.maximum(
        jnp.dot(f_ref[...], nw1_ref[...], preferred_element_type=jnp.float32)
        + nb1_ref[...], 0.0)
    xg = _ln(jnp.dot(h, nw2_ref[...], preferred_element_type=jnp.float32)
             + nb2_ref[...], ng_ref[...], nbe_ref[...])
    xg_ref[...] = xg
    h = jnp.maximum(
        jnp.dot(ef_ref[...], ew1_ref[...], preferred_element_type=jnp.float32)
        + eb1_ref[...], 0.0)
    ee = _ln(jnp.dot(h, ew2_ref[...], preferred_element_type=jnp.float32)
             + eb2_ref[...], eg_ref[...], ebe_ref[...])
    h = jnp.maximum(
        jnp.dot(xg, wa_ref[...], preferred_element_type=jnp.float32)
        + jnp.dot(ee, wc_ref[...], preferred_element_type=jnp.float32)
        + gb1_ref[...], 0.0)
    m_ref[...] = _ln(jnp.dot(h, gw2_ref[...], preferred_element_type=jnp.float32)
                     + gb2_ref[...], gg_ref[...], gbe_ref[...])


def _enc1_call(feats_p, enc_ef_p, params):
    pn, pe, pg = params['enc_node'], params['enc_edge'], params['enc_gn_e']
    nw1 = jnp.zeros((80, 128), jnp.float32).at[:FEAT].set(pn['l1']['w'])
    ew1 = jnp.zeros((8, 128), jnp.float32).at[:3].set(pe['l1']['w'])
    gw1 = pg['l1']['w']
    nb = pl.BlockSpec(None, lambda: (0, 0))
    out_sh = jax.ShapeDtypeStruct((N_GRID, 128), jnp.float32)
    return pl.pallas_call(
        _enc1_body,
        in_specs=[nb] * 21,
        out_specs=[nb, nb],
        out_shape=[out_sh, out_sh],
    )(feats_p, enc_ef_p,
      nw1, pn['l1']['b'][None], pn['l2']['w'], pn['l2']['b'][None],
      pn['g'][None], pn['be'][None],
      ew1, pe['l1']['b'][None], pe['l2']['w'], pe['l2']['b'][None],
      pe['g'][None], pe['be'][None],
      gw1[:ND], gw1[2 * ND:], pg['l1']['b'][None], pg['l2']['w'],
      pg['l2']['b'][None], pg['g'][None], pg['be'][None])


def _dec_body(xg_ref, gd_ref, ef_ref, f_ref,
              ew1_ref, eb1_ref, ew2_ref, eb2_ref, eg_ref, ebe_ref,
              wb_ref, wc_ref, gb1_ref, gw2_ref, gb2_ref, gg_ref, gbe_ref,
              wna_ref, wnb_ref, nb1_ref, nw2_ref, nb2_ref, ng_ref, nbe_ref,
              ow1_ref, ob1_ref, ow2_ref, ob2_ref,
              out_ref):
    xg = xg_ref[...]
    h = jnp.maximum(
        jnp.dot(ef_ref[...], ew1_ref[...], preferred_element_type=jnp.float32)
        + eb1_ref[...], 0.0)
    ed = _ln(jnp.dot(h, ew2_ref[...], preferred_element_type=jnp.float32)
             + eb2_ref[...], eg_ref[...], ebe_ref[...])
    h = jnp.maximum(
        gd_ref[...]
        + jnp.dot(xg, wb_ref[...], preferred_element_type=jnp.float32)
        + jnp.dot(ed, wc_ref[...], preferred_element_type=jnp.float32)
        + gb1_ref[...], 0.0)
    m = _ln(jnp.dot(h, gw2_ref[...], preferred_element_type=jnp.float32)
            + gb2_ref[...], gg_ref[...], gbe_ref[...])
    h = jnp.maximum(
        jnp.dot(xg, wna_ref[...], preferred_element_type=jnp.float32)
        + jnp.dot(m, wnb_ref[...], preferred_element_type=jnp.float32)
        + nb1_ref[...], 0.0)
    x_out = _ln(jnp.dot(h, nw2_ref[...], preferred_element_type=jnp.float32)
                + nb2_ref[...], ng_ref[...], nbe_ref[...])
    d1 = jnp.maximum(
        jnp.dot(x_out, ow1_ref[...], preferred_element_type=jnp.float32)
        + ob1_ref[...], 0.0)
    delta = jnp.dot(d1, ow2_ref[...], preferred_element_type=jnp.float32) + ob2_ref[...]
    out_ref[...] = f_ref[...] + delta


def _dec_call(x_grid, gdec, dec_ef_p, feats_p, params):
    pe, pg, pn, po = (params['dec_edge'], params['dec_gn_e'],
                      params['dec_gn_n'], params['dec_out'])
    ew1 = jnp.zeros((8, 128), jnp.float32).at[:3].set(pe['l1']['w'])
    gw1 = pg['l1']['w']
    nw1 = pn['l1']['w']
    ow2 = jnp.zeros((HDD, 80), jnp.float32).at[:, :FEAT].set(po['l2']['w'])
    ob2 = jnp.zeros((1, 80), jnp.float32).at[0, :FEAT].set(po['l2']['b'])
    nb = pl.BlockSpec(None, lambda: (0, 0))
    return pl.pallas_call(
        _dec_body,
        in_specs=[nb] * 28,
        out_specs=nb,
        out_shape=jax.ShapeDtypeStruct((N_GRID, 80), jnp.float32),
    )(x_grid, gdec, dec_ef_p, feats_p,
      ew1, pe['l1']['b'][None], pe['l2']['w'], pe['l2']['b'][None],
      pe['g'][None], pe['be'][None],
      gw1[ND:2 * ND], gw1[2 * ND:], pg['l1']['b'][None], pg['l2']['w'],
      pg['l2']['b'][None], pg['g'][None], pg['be'][None],
      nw1[:ND], nw1[ND:], pn['l1']['b'][None], pn['l2']['w'],
      pn['l2']['b'][None], pn['g'][None], pn['be'][None],
      po['l1']['w'], po['l1']['b'][None], ow2, ob2)


_NW = 32
_NMT = NM_PAD // 16


def _make_gather_pair(nchunks):
    epw = nchunks * 128
    e_tot = _NW * epw
    mesh = plsc.VectorSubcoreMesh(core_axis_name="c", subcore_axis_name="s")
    out_sh = jax.ShapeDtypeStruct((e_tot, 128), jnp.float32)

    depth = min(3, nchunks)

    @functools.partial(
        pl.kernel, out_type=out_sh, mesh=mesh,
        scratch_types=[
            pltpu.VMEM((nchunks, 128), jnp.int32),
            pltpu.VMEM((nchunks, 128), jnp.int32),
            [pltpu.VMEM((128, 128), jnp.float32)] * depth,
            [pltpu.VMEM((128, 128), jnp.float32)] * depth,
            [pltpu.SemaphoreType.DMA] * depth,
            [pltpu.SemaphoreType.DMA] * depth,
            pltpu.SemaphoreType.DMA,
        ])
    def g(xa, xb, ps3, pd3, gs, ia, ib, ab, bb, sga, sgb, sw):
        sid = lax.axis_index("s")
        wid = sid * 2 + lax.axis_index("c")
        base = wid * epw
        pltpu.sync_copy(ps3.at[wid], ia)
        pltpu.sync_copy(pd3.at[wid], ib)
        gath = [None] * nchunks
        for j in range(depth):
            gath[j] = (pltpu.async_copy(xa.at[ia.at[j]], ab[j], sga[j]),
                       pltpu.async_copy(xb.at[ib.at[j]], bb[j], sgb[j]))
        for j in range(nchunks):
            p = j % depth
            ca, cb = gath[j]
            ca.wait()
            cb.wait()
            abuf = ab[p]
            bbuf = bb[p]

            def add_row(r, _):
                for c8 in range(8):
                    sl = pl.ds(c8 * 16, 16)
                    abuf[r, sl] = abuf[r, sl] + bbuf[r, sl]
                return 0

            lax.fori_loop(0, 128, add_row, 0, unroll=2)
            pltpu.async_copy(abuf, gs.at[pl.ds(base + j * 128, 128)],
                             sw).wait()
            if j + depth < nchunks:
                gath[j + depth] = (
                    pltpu.async_copy(xa.at[ia.at[j + depth]], ab[p], sga[p]),
                    pltpu.async_copy(xb.at[ib.at[j + depth]], bb[p], sgb[p]))

    return g


def _oh_scatter_body(d_ref, m_ref, out_ref):
    k = pl.program_id(0)
    base = k * _MT
    iota = lax.broadcasted_iota(jnp.int32, (N_GRID, _MT), 1) + base
    b = (d_ref[...] == iota).astype(jnp.float32)
    out_ref[...] = lax.dot_general(
        b, m_ref[...], (((0,), (0,)), ((), ())),
        preferred_element_type=jnp.float32)


def _oh_scatter(d2, m):
    nb = pl.BlockSpec(None, lambda k: (0, 0))
    return pl.pallas_call(
        _oh_scatter_body,
        grid=(NM_PAD // _MT,),
        in_specs=[nb, nb],
        out_specs=pl.BlockSpec((_MT, 128), lambda k: (k, 0)),
        out_shape=jax.ShapeDtypeStruct((NM_PAD, 128), jnp.float32),
    )(d2, m)


def _oh_gather_body(d_ref, t_ref, out_ref):
    k = pl.program_id(0)
    base = k * _MT
    iota = lax.broadcasted_iota(jnp.int32, (N_GRID, _MT), 1) + base
    b = (d_ref[...] == iota).astype(jnp.float32)
    part = jnp.dot(b, t_ref[...], preferred_element_type=jnp.float32)

    @pl.when(k == 0)
    def _():
        out_ref[...] = part

    @pl.when(k > 0)
    def _():
        out_ref[...] = out_ref[...] + part


def _oh_gather(d2, table):
    nb = pl.BlockSpec(None, lambda k: (0, 0))
    return pl.pallas_call(
        _oh_gather_body,
        grid=(NM_PAD // _MT,),
        in_specs=[nb, pl.BlockSpec((_MT, 128), lambda k: (k, 0))],
        out_specs=pl.BlockSpec((N_GRID, 128), lambda k: (0, 0)),
        out_shape=jax.ShapeDtypeStruct((N_GRID, 128), jnp.float32),
    )(d2, table)


def _make_scatter(nchunks):
    epw = nchunks * 128
    mesh = plsc.VectorSubcoreMesh(core_axis_name="c", subcore_axis_name="s")
    out_sh = jax.ShapeDtypeStruct((2, NM_PAD, 128), jnp.float32)

    @functools.partial(
        pl.kernel, out_type=out_sh, mesh=mesh,
        scratch_types=[
            pltpu.VMEM((nchunks, 128), jnp.int32),
            pltpu.VMEM((128, 128), jnp.float32),
            pltpu.VMEM((128, 128), jnp.float32),
            pltpu.VMEM_SHARED((NM_PAD, 128), jnp.float32),
            pltpu.SemaphoreType.DMA,
            pltpu.SemaphoreType.DMA,
        ])
    def s(e_hbm, idx3, zeros_hbm, out, ib, e0, e1, acc, sld, ssc):
        c = lax.axis_index("c")
        sid = lax.axis_index("s")
        wid = sid * 2 + c
        base = wid * epw
        cz = pltpu.async_copy(zeros_hbm.at[pl.ds(sid * _NMT, _NMT)],
                              acc.at[pl.ds(sid * _NMT, _NMT)], ssc)
        pltpu.sync_copy(idx3.at[wid], ib)
        eb = [e0, e1]
        ld = [None] * nchunks
        sc = [None] * nchunks
        ld[0] = pltpu.async_copy(e_hbm.at[pl.ds(base, 128)], e0, sld)
        cz.wait()
        plsc.subcore_barrier()
        for j in range(nchunks):
            p = j % 2
            ld[j].wait()
            if j + 1 < nchunks:
                if j >= 1:
                    sc[j - 1].wait()
                ld[j + 1] = pltpu.async_copy(
                    e_hbm.at[pl.ds(base + (j + 1) * 128, 128)], eb[1 - p], sld)
            sc[j] = pltpu.async_copy(eb[p], acc.at[ib.at[j]], ssc, add=True)
        for j in (nchunks - 2, nchunks - 1):
            if j >= 0 and sc[j] is not None:
                sc[j].wait()
        plsc.subcore_barrier()
        pltpu.sync_copy(acc.at[pl.ds(sid * _NMT, _NMT)],
                        out.at[c, pl.ds(sid * _NMT, _NMT)])

    return s


_gather_proc = _make_gather_pair(9)
_scatter_proc = _make_scatter(9)


@jax.jit
def _run(features, params, enc_ef, proc_ef, dec_ef, enc_edges, proc_edges,
         dec_edges):
    feats_p = jnp.zeros((N_GRID, 80), jnp.float32).at[:, :FEAT].set(
        features.reshape(N_GRID, FEAT))
    enc_ef_p = jnp.zeros((N_GRID, 8), jnp.float32).at[:, :3].set(enc_ef)
    dec_ef_p = jnp.zeros((N_GRID, 8), jnp.float32).at[:, :3].set(dec_ef)
    pf_pad = jnp.zeros((EP_PAD, 8), jnp.float32).at[:E_PROC, :3].set(proc_ef)

    ps = proc_edges[0]
    pd = proc_edges[1]
    ps3 = jnp.pad(ps, (0, EP_PAD - E_PROC)).reshape(_NW, 9, 128)
    pd3 = jnp.pad(pd, (0, EP_PAD - E_PROC)).reshape(_NW, 9, 128)
    pds3 = jnp.pad(pd, (0, EP_PAD - E_PROC),
                   constant_values=N_MESH).reshape(_NW, 9, 128)
    zeros_hbm = jnp.zeros((NM_PAD, 128), jnp.float32)

    x_grid, m = _enc1_call(feats_p, enc_ef_p, params)
    agg_enc = _oh_scatter(enc_edges[1][:, None], m)
    blk0 = params['blocks'][0]
    w1n = blk0['e']['l1']['w']
    zeros_x = jnp.zeros((NM_PAD, 128), jnp.float32)
    x, xa, xb = _ea_call(zeros_x, agg_enc, zeros_hbm, params['enc_gn_n'],
                         w1n[:ND], w1n[ND:2 * ND])

    e = _ef_call(pf_pad, params['proc_edge'])

    nb = len(params['blocks'])
    gw1 = params['dec_gn_e']['l1']['w']
    for k, blk in enumerate(params['blocks']):
        s = _gather_proc(xa, xb, ps3, pd3)
        e = _c_call(s, e, blk['e'])
        agg = _scatter_proc(e, pds3, zeros_hbm)
        if k + 1 < nb:
            w1n = params['blocks'][k + 1]['e']['l1']['w']
            wa_next, wb_next = w1n[:ND], w1n[ND:2 * ND]
        else:
            wa_next, wb_next = gw1[:ND], gw1[ND:2 * ND]
        x, xa, xb = _ea_call(x, agg[0], agg[1], blk['n'], wa_next, wb_next)

    gdec = _oh_gather(dec_edges[0][:, None], xa)
    out = _dec_call(x_grid, gdec, dec_ef_p, feats_p, params)
    return out[:, :FEAT].reshape(1, N_GRID, FEAT)


def kernel(features, params, enc_ef, proc_ef, dec_ef, enc_edges, proc_edges,
           dec_edges):
    return _run(features, params, enc_ef, proc_ef, dec_ef, enc_edges,
                proc_edges, dec_edges)
